# baseline (speedup 1.0000x reference)
import functools

import jax
import jax.numpy as jnp
from jax import lax
from jax.experimental import pallas as pl
from jax.experimental.pallas import tpu as pltpu
from jax.experimental.pallas import tpu_sc as plsc

N_EMB = 100000
D = 128
N_TRIP = 100000

NC = 2
NS = 16
NW = NC * NS
BLK = 128
NB_PER = 25
N_PAD = NW * NB_PER * BLK
LANES = 16
CHUNKS = D // LANES

_mesh = plsc.VectorSubcoreMesh(core_axis_name="c", subcore_axis_name="s")


@functools.partial(
    pl.kernel,
    mesh=_mesh,
    out_type=jax.ShapeDtypeStruct((N_PAD, LANES), jnp.float32),
    scratch_types=[
        pltpu.VMEM((BLK,), jnp.int32),
        pltpu.VMEM((BLK,), jnp.int32),
        pltpu.VMEM((BLK,), jnp.int32),
        pltpu.VMEM((BLK,), jnp.int32),
        pltpu.VMEM((BLK,), jnp.int32),
        pltpu.VMEM((BLK,), jnp.int32),
        pltpu.VMEM((BLK, D), jnp.float32),
        pltpu.VMEM((BLK, D), jnp.float32),
        pltpu.VMEM((BLK, D), jnp.float32),
        pltpu.VMEM((BLK, D), jnp.float32),
        pltpu.VMEM((BLK, D), jnp.float32),
        pltpu.VMEM((BLK, D), jnp.float32),
        pltpu.VMEM((BLK, LANES), jnp.float32),
        pltpu.SemaphoreType.DMA,
        pltpu.SemaphoreType.DMA,
    ],
)
def _sc_partials(emb, t0, t1, t2, out,
                 ia0, it0, ii0, ia1, it1, ii1,
                 a0, tb0, ib0, a1, tb1, ib1, xp, sem0, sem1):
    wid = lax.axis_index("s") * NC + lax.axis_index("c")
    base0 = wid * (NB_PER * BLK)

    bufs = ((ia0, it0, ii0, a0, tb0, ib0, sem0),
            (ia1, it1, ii1, a1, tb1, ib1, sem1))

    def fetch(blk, s):
        ia, it, ii, ab, tb, ib, sem = bufs[s]
        off = base0 + blk * BLK
        pltpu.sync_copy(t0.at[pl.ds(off, BLK)], ia)
        pltpu.sync_copy(t1.at[pl.ds(off, BLK)], it)
        pltpu.sync_copy(t2.at[pl.ds(off, BLK)], ii)
        pltpu.make_async_copy(emb.at[ia], ab, sem).start()
        pltpu.make_async_copy(emb.at[it], tb, sem).start()
        pltpu.make_async_copy(emb.at[ii], ib, sem).start()

    def drain(s):
        ia, it, ii, ab, tb, ib, sem = bufs[s]
        pltpu.make_async_copy(emb.at[ia], ab, sem).wait()
        pltpu.make_async_copy(emb.at[it], tb, sem).wait()
        pltpu.make_async_copy(emb.at[ii], ib, sem).wait()

    def compute(blk, s):
        _, _, _, ab, tb, ib, _ = bufs[s]

        def triplet(j, carry):
            accp = jnp.zeros((LANES,), jnp.float32)
            accn = jnp.zeros((LANES,), jnp.float32)
            for c in range(CHUNKS):
                sl = pl.ds(c * LANES, LANES)
                av = ab[j, sl]
                tv = tb[j, sl]
                iv = ib[j, sl]
                d1 = tv - av
                d2 = iv - av
                d3 = iv - tv
                accp = accp + d1 * d1
                accn = accn + d2 * d2 + d3 * d3
            xp[j, :] = accp - 0.5 * accn
            return carry

        lax.fori_loop(0, BLK, triplet, 0)
        off = base0 + blk * BLK
        pltpu.sync_copy(xp, out.at[pl.ds(off, BLK), :])

    fetch(0, 0)
    fetch(1, 1)
    drain(0)
    compute(0, 0)

    def outer(g, carry):
        blk1 = 1 + 2 * g
        fetch(blk1 + 1, 0)
        drain(1)
        compute(blk1, 1)
        blk2 = blk1 + 1

        @pl.when(blk2 + 1 < NB_PER)
        def _():
            fetch(blk2 + 1, 1)

        drain(0)
        compute(blk2, 0)
        return carry

    lax.fori_loop(0, (NB_PER - 1) // 2, outer, 0)


_TC_BLK = 2048


def _tc_reduce_body(x_ref, out_ref):
    i = pl.program_id(0)
    x = x_ref[...]
    s = jnp.sum(x, axis=1, keepdims=True)
    sp = jnp.maximum(s, 0.0) + jnp.log1p(jnp.exp(-jnp.abs(s)))
    rows = lax.broadcasted_iota(jnp.int32, (_TC_BLK, 1), 0) + i * _TC_BLK
    part = jnp.sum(jnp.where(rows < N_TRIP, sp, 0.0), keepdims=True).reshape(1, 1) / N_TRIP

    @pl.when(i == 0)
    def _():
        out_ref[...] = part

    @pl.when(i > 0)
    def _():
        out_ref[...] = out_ref[...] + part


_tc_reduce = pl.pallas_call(
    _tc_reduce_body,
    grid=(N_PAD // _TC_BLK,),
    in_specs=[pl.BlockSpec((_TC_BLK, LANES), lambda i: (i, 0))],
    out_specs=pl.BlockSpec((1, 1), lambda i: (0, 0)),
    out_shape=jax.ShapeDtypeStruct((1, 1), jnp.float32),
)


def kernel(inputs, targets, T):
    del targets
    t_pad = jnp.pad(T, ((0, 0), (0, N_PAD - T.shape[1])))
    xpart = _sc_partials(inputs, t_pad[0], t_pad[1], t_pad[2])
    return _tc_reduce(xpart)[0, 0]


# speedup vs baseline: 1.0133x; 1.0133x over previous
import functools

import jax
import jax.numpy as jnp
from jax import lax
from jax.experimental import pallas as pl
from jax.experimental.pallas import tpu as pltpu
from jax.experimental.pallas import tpu_sc as plsc

N_EMB = 100000
D = 128
N_TRIP = 100000

NC = 2
NS = 16
NW = NC * NS
BLK = 128
NB_PER = 25
N_PAD = NW * NB_PER * BLK
LANES = 16
CHUNKS = D // LANES

_mesh = plsc.VectorSubcoreMesh(core_axis_name="c", subcore_axis_name="s")


@functools.partial(
    pl.kernel,
    mesh=_mesh,
    out_type=jax.ShapeDtypeStruct((N_PAD, LANES), jnp.float32),
    scratch_types=[
        pltpu.VMEM((BLK,), jnp.int32),
        pltpu.VMEM((BLK,), jnp.int32),
        pltpu.VMEM((BLK,), jnp.int32),
        pltpu.VMEM((BLK,), jnp.int32),
        pltpu.VMEM((BLK,), jnp.int32),
        pltpu.VMEM((BLK,), jnp.int32),
        pltpu.VMEM((BLK, D), jnp.float32),
        pltpu.VMEM((BLK, D), jnp.float32),
        pltpu.VMEM((BLK, D), jnp.float32),
        pltpu.VMEM((BLK, D), jnp.float32),
        pltpu.VMEM((BLK, D), jnp.float32),
        pltpu.VMEM((BLK, D), jnp.float32),
        pltpu.VMEM((BLK, LANES), jnp.float32),
        pltpu.SemaphoreType.DMA,
        pltpu.SemaphoreType.DMA,
        pltpu.SemaphoreType.DMA,
        pltpu.SemaphoreType.DMA,
        pltpu.SemaphoreType.DMA,
    ],
)
def _sc_partials(emb, t0, t1, t2, out,
                 ia0, it0, ii0, ia1, it1, ii1,
                 a0, tb0, ib0, a1, tb1, ib1,
                 xp, sem0, sem1, si0, si1, semo):
    wid = lax.axis_index("s") * NC + lax.axis_index("c")
    base0 = wid * (NB_PER * BLK)

    bufs = ((ia0, it0, ii0, a0, tb0, ib0, sem0, si0),
            (ia1, it1, ii1, a1, tb1, ib1, sem1, si1))

    def idx_fetch(blk, s):
        ia, it, ii, _, _, _, _, si = bufs[s]
        off = base0 + blk * BLK
        pltpu.make_async_copy(t0.at[pl.ds(off, BLK)], ia, si).start()
        pltpu.make_async_copy(t1.at[pl.ds(off, BLK)], it, si).start()
        pltpu.make_async_copy(t2.at[pl.ds(off, BLK)], ii, si).start()

    def idx_wait(s):
        ia, it, ii, _, _, _, _, si = bufs[s]
        pltpu.make_async_copy(t0.at[pl.ds(base0, BLK)], ia, si).wait()
        pltpu.make_async_copy(t0.at[pl.ds(base0, BLK)], it, si).wait()
        pltpu.make_async_copy(t0.at[pl.ds(base0, BLK)], ii, si).wait()

    def fire(s):
        ia, it, ii, ab, tb, ib, sem, _ = bufs[s]
        pltpu.make_async_copy(emb.at[ia], ab, sem).start()
        pltpu.make_async_copy(emb.at[it], tb, sem).start()
        pltpu.make_async_copy(emb.at[ii], ib, sem).start()

    def drain(s):
        ia, it, ii, ab, tb, ib, sem, _ = bufs[s]
        pltpu.make_async_copy(emb.at[ia], ab, sem).wait()
        pltpu.make_async_copy(emb.at[it], tb, sem).wait()
        pltpu.make_async_copy(emb.at[ii], ib, sem).wait()

    def wait_out():
        pltpu.make_async_copy(xp, out.at[pl.ds(base0, BLK), :], semo).wait()

    def compute(blk, s):
        _, _, _, ab, tb, ib, _, _ = bufs[s]

        def triplet(j, carry):
            acc1 = jnp.zeros((LANES,), jnp.float32)
            acc12 = jnp.zeros((LANES,), jnp.float32)
            acc2 = jnp.zeros((LANES,), jnp.float32)
            for c in range(CHUNKS):
                sl = pl.ds(c * LANES, LANES)
                av = ab[j, sl]
                tv = tb[j, sl]
                iv = ib[j, sl]
                d1 = tv - av
                d2 = iv - av
                acc1 = acc1 + d1 * d1
                acc12 = acc12 + d1 * d2
                acc2 = acc2 + d2 * d2
            xp[j, :] = 0.5 * acc1 + acc12 - acc2
            return carry

        lax.fori_loop(0, BLK, triplet, 0)
        off = base0 + blk * BLK
        pltpu.make_async_copy(xp, out.at[pl.ds(off, BLK), :], semo).start()

    idx_fetch(0, 0)
    idx_fetch(1, 1)
    idx_wait(0)
    fire(0)
    idx_wait(1)
    fire(1)
    drain(0)
    idx_fetch(2, 0)
    compute(0, 0)

    def outer(g, carry):
        blk1 = 1 + 2 * g
        blk2 = blk1 + 1

        idx_wait(0)
        fire(0)
        drain(1)

        @pl.when(blk2 + 1 < NB_PER)
        def _():
            idx_fetch(blk2 + 1, 1)

        wait_out()
        compute(blk1, 1)

        @pl.when(blk2 + 1 < NB_PER)
        def _():
            idx_wait(1)
            fire(1)

        drain(0)

        @pl.when(blk2 + 2 < NB_PER)
        def _():
            idx_fetch(blk2 + 2, 0)

        wait_out()
        compute(blk2, 0)
        return carry

    lax.fori_loop(0, (NB_PER - 1) // 2, outer, 0)
    wait_out()


_TC_BLK = 2048


def _tc_reduce_body(x_ref, out_ref):
    i = pl.program_id(0)
    x = x_ref[...]
    s = jnp.sum(x, axis=1, keepdims=True)
    sp = jnp.maximum(s, 0.0) + jnp.log1p(jnp.exp(-jnp.abs(s)))
    rows = lax.broadcasted_iota(jnp.int32, (_TC_BLK, 1), 0) + i * _TC_BLK
    part = jnp.sum(jnp.where(rows < N_TRIP, sp, 0.0), keepdims=True).reshape(1, 1) / N_TRIP

    @pl.when(i == 0)
    def _():
        out_ref[...] = part

    @pl.when(i > 0)
    def _():
        out_ref[...] = out_ref[...] + part


_tc_reduce = pl.pallas_call(
    _tc_reduce_body,
    grid=(N_PAD // _TC_BLK,),
    in_specs=[pl.BlockSpec((_TC_BLK, LANES), lambda i: (i, 0))],
    out_specs=pl.BlockSpec((1, 1), lambda i: (0, 0)),
    out_shape=jax.ShapeDtypeStruct((1, 1), jnp.float32),
)


def kernel(inputs, targets, T):
    del targets
    t_pad = jnp.pad(T, ((0, 0), (0, N_PAD - T.shape[1])))
    xpart = _sc_partials(inputs, t_pad[0], t_pad[1], t_pad[2])
    return _tc_reduce(xpart)[0, 0]


# speedup vs baseline: 1.1305x; 1.1156x over previous
import functools

import jax
import jax.numpy as jnp
from jax import lax
from jax.experimental import pallas as pl
from jax.experimental.pallas import tpu as pltpu
from jax.experimental.pallas import tpu_sc as plsc

N_EMB = 100000
D = 128
N_TRIP = 100000

NC = 2
NS = 16
NW = NC * NS
BLK = 128
NB_PER = 25
N_PAD = NW * NB_PER * BLK
LANES = 16
CHUNKS = D // LANES

_mesh = plsc.VectorSubcoreMesh(core_axis_name="c", subcore_axis_name="s")


@functools.partial(
    pl.kernel,
    mesh=_mesh,
    out_type=jax.ShapeDtypeStruct((N_PAD, LANES), jnp.float32),
    scratch_types=[
        pltpu.VMEM((BLK,), jnp.int32),
        pltpu.VMEM((BLK,), jnp.int32),
        pltpu.VMEM((BLK,), jnp.int32),
        pltpu.VMEM((BLK,), jnp.int32),
        pltpu.VMEM((BLK,), jnp.int32),
        pltpu.VMEM((BLK,), jnp.int32),
        pltpu.VMEM((BLK, D), jnp.float32),
        pltpu.VMEM((BLK, D), jnp.float32),
        pltpu.VMEM((BLK, D), jnp.float32),
        pltpu.VMEM((BLK, D), jnp.float32),
        pltpu.VMEM((BLK, D), jnp.float32),
        pltpu.VMEM((BLK, D), jnp.float32),
        pltpu.VMEM((BLK, LANES), jnp.float32),
        pltpu.SemaphoreType.DMA,
        pltpu.SemaphoreType.DMA,
        pltpu.SemaphoreType.DMA,
        pltpu.SemaphoreType.DMA,
        pltpu.SemaphoreType.DMA,
    ],
)
def _sc_partials(emb, t0, t1, t2, out,
                 ia0, it0, ii0, ia1, it1, ii1,
                 a0, tb0, ib0, a1, tb1, ib1,
                 xp, sem0, sem1, si0, si1, semo):
    wid = lax.axis_index("s") * NC + (1 - lax.axis_index("c"))
    base0 = wid * (NB_PER * BLK)

    bufs = ((ia0, it0, ii0, a0, tb0, ib0, sem0, si0),
            (ia1, it1, ii1, a1, tb1, ib1, sem1, si1))

    def idx_fetch(blk, s):
        ia, it, ii, _, _, _, _, si = bufs[s]
        off = base0 + blk * BLK
        pltpu.make_async_copy(t0.at[pl.ds(off, BLK)], ia, si).start()
        pltpu.make_async_copy(t1.at[pl.ds(off, BLK)], it, si).start()
        pltpu.make_async_copy(t2.at[pl.ds(off, BLK)], ii, si).start()

    def idx_wait(s):
        ia, it, ii, _, _, _, _, si = bufs[s]
        pltpu.make_async_copy(t0.at[pl.ds(base0, BLK)], ia, si).wait()
        pltpu.make_async_copy(t0.at[pl.ds(base0, BLK)], it, si).wait()
        pltpu.make_async_copy(t0.at[pl.ds(base0, BLK)], ii, si).wait()

    def fire(s):
        ia, it, ii, ab, tb, ib, sem, _ = bufs[s]
        pltpu.make_async_copy(emb.at[ia], ab, sem).start()
        pltpu.make_async_copy(emb.at[it], tb, sem).start()
        pltpu.make_async_copy(emb.at[ii], ib, sem).start()

    def drain(s):
        ia, it, ii, ab, tb, ib, sem, _ = bufs[s]
        pltpu.make_async_copy(emb.at[ia], ab, sem).wait()
        pltpu.make_async_copy(emb.at[it], tb, sem).wait()
        pltpu.make_async_copy(emb.at[ii], ib, sem).wait()

    def wait_out():
        pltpu.make_async_copy(xp, out.at[pl.ds(base0, BLK), :], semo).wait()

    def compute(blk, s):
        _, _, _, ab, tb, ib, _, _ = bufs[s]

        def triplet(j, carry):
            acc1 = jnp.zeros((LANES,), jnp.float32)
            acc12 = jnp.zeros((LANES,), jnp.float32)
            acc2 = jnp.zeros((LANES,), jnp.float32)
            for c in range(CHUNKS):
                sl = pl.ds(c * LANES, LANES)
                av = ab[j, sl]
                tv = tb[j, sl]
                iv = ib[j, sl]
                d1 = tv - av
                d2 = iv - av
                acc1 = acc1 + d1 * d1
                acc12 = acc12 + d1 * d2
                acc2 = acc2 + d2 * d2
            xp[j, :] = 0.5 * acc1 + acc12 - acc2
            return carry

        lax.fori_loop(0, BLK, triplet, 0)
        off = base0 + blk * BLK
        pltpu.make_async_copy(xp, out.at[pl.ds(off, BLK), :], semo).start()

    idx_fetch(0, 0)
    idx_fetch(1, 1)
    idx_wait(0)
    fire(0)
    idx_wait(1)
    fire(1)
    drain(0)
    idx_fetch(2, 0)
    compute(0, 0)

    def outer(g, carry):
        blk1 = 1 + 2 * g
        blk2 = blk1 + 1

        idx_wait(0)
        fire(0)
        drain(1)

        @pl.when(blk2 + 1 < NB_PER)
        def _():
            idx_fetch(blk2 + 1, 1)

        wait_out()
        compute(blk1, 1)

        @pl.when(blk2 + 1 < NB_PER)
        def _():
            idx_wait(1)
            fire(1)

        drain(0)

        @pl.when(blk2 + 2 < NB_PER)
        def _():
            idx_fetch(blk2 + 2, 0)

        wait_out()
        compute(blk2, 0)
        return carry

    lax.fori_loop(0, (NB_PER - 1) // 2, outer, 0)
    wait_out()


def _tc_reduce_body(x_ref, out_ref):
    x = x_ref[...]
    ones = jnp.ones((16, 1), jnp.float32)
    g = lax.broadcasted_iota(jnp.int32, (256, 16), 0) // 16 == \
        lax.broadcasted_iota(jnp.int32, (256, 16), 1)
    s = jax.lax.dot_general(x, g.astype(jnp.float32),
                            (((1,), (0,)), ((), ())),
                            preferred_element_type=jnp.float32)
    sp = jnp.maximum(s, 0.0) + jnp.log1p(jnp.exp(-jnp.abs(s)))
    ids = lax.broadcasted_iota(jnp.int32, (N_PAD // 16, 16), 0) * 16 + \
        lax.broadcasted_iota(jnp.int32, (N_PAD // 16, 16), 1)
    total = jnp.sum(jnp.where(ids < N_TRIP, sp, 0.0), keepdims=True)
    out_ref[...] = total.reshape(1, 1) / N_TRIP


_tc_reduce = pl.pallas_call(
    _tc_reduce_body,
    out_shape=jax.ShapeDtypeStruct((1, 1), jnp.float32),
)


def kernel(inputs, targets, T):
    del targets
    t_pad = jnp.pad(T, ((0, 0), (0, N_PAD - T.shape[1])))
    xpart = _sc_partials(inputs, t_pad[0], t_pad[1], t_pad[2])
    return _tc_reduce(xpart.reshape(N_PAD // 16, 256))[0, 0]


# speedup vs baseline: 2.9371x; 2.5982x over previous
import functools

import jax
import jax.numpy as jnp
from jax import lax
from jax.experimental import pallas as pl
from jax.experimental.pallas import tpu as pltpu
from jax.experimental.pallas import tpu_sc as plsc

N_EMB = 100000
D = 128
N_TRIP = 100000

NC = 2
NS = 16
NW = NC * NS
BLK = 112
GRP = 4
NB_PER = 28
NPAIR = NB_PER // 2
N_PAD = NW * NB_PER * BLK
LANES = 16
CHUNKS = D // LANES

_mesh = plsc.VectorSubcoreMesh(core_axis_name="c", subcore_axis_name="s")


@functools.partial(
    pl.kernel,
    mesh=_mesh,
    out_type=jax.ShapeDtypeStruct((N_PAD * LANES,), jnp.float32),
    scratch_types=[
        pltpu.VMEM((2 * BLK,), jnp.int32),
        pltpu.VMEM((2 * BLK,), jnp.int32),
        pltpu.VMEM((2 * BLK,), jnp.int32),
        pltpu.VMEM((2 * BLK,), jnp.int32),
        pltpu.VMEM((2 * BLK,), jnp.int32),
        pltpu.VMEM((2 * BLK,), jnp.int32),
        pltpu.VMEM((BLK, D), jnp.float32),
        pltpu.VMEM((BLK, D), jnp.float32),
        pltpu.VMEM((BLK, D), jnp.float32),
        pltpu.VMEM((BLK, D), jnp.float32),
        pltpu.VMEM((BLK, D), jnp.float32),
        pltpu.VMEM((BLK, D), jnp.float32),
        pltpu.VMEM((GRP * BLK * LANES,), jnp.float32),
        pltpu.VMEM((GRP * BLK * LANES,), jnp.float32),
        pltpu.SemaphoreType.DMA,
        pltpu.SemaphoreType.DMA,
        pltpu.SemaphoreType.DMA,
        pltpu.SemaphoreType.DMA,
        pltpu.SemaphoreType.DMA,
        pltpu.SemaphoreType.DMA,
    ],
)
def _sc_partials(emb, t0, t1, t2, out,
                 ia0, it0, ii0, ia1, it1, ii1,
                 a0, tb0, ib0, a1, tb1, ib1,
                 xpa, xpb, semr0, semr1, si0, si1, semoa, semob):
    wid = lax.axis_index("s") * NC + lax.axis_index("c")
    base0 = wid * (NB_PER * BLK)

    idxs = ((ia0, it0, ii0, si0), (ia1, it1, ii1, si1))
    rows = ((a0, tb0, ib0, semr0), (a1, tb1, ib1, semr1))
    xps = ((xpa, semoa), (xpb, semob))

    def idx_fetch(pair, iset):
        ia, it, ii, si = idxs[iset]
        off = base0 + pair * (2 * BLK)
        pltpu.make_async_copy(t0.at[pl.ds(off, 2 * BLK)], ia, si).start()
        pltpu.make_async_copy(t1.at[pl.ds(off, 2 * BLK)], it, si).start()
        pltpu.make_async_copy(t2.at[pl.ds(off, 2 * BLK)], ii, si).start()

    def idx_wait(iset):
        ia, it, ii, si = idxs[iset]
        pltpu.make_async_copy(t0.at[pl.ds(base0, 2 * BLK)], ia, si).wait()
        pltpu.make_async_copy(t0.at[pl.ds(base0, 2 * BLK)], it, si).wait()
        pltpu.make_async_copy(t0.at[pl.ds(base0, 2 * BLK)], ii, si).wait()

    def fire(rset, iset, half):
        ia, it, ii, _ = idxs[iset]
        ab, tb, ib, semr = rows[rset]
        sl = pl.ds(half * BLK, BLK)
        pltpu.make_async_copy(emb.at[ia.at[sl]], ab, semr).start()
        pltpu.make_async_copy(emb.at[it.at[sl]], tb, semr).start()
        pltpu.make_async_copy(emb.at[ii.at[sl]], ib, semr).start()

    def drain(rset):
        ia, _, _, _ = idxs[0]
        ab, tb, ib, semr = rows[rset]
        sl = pl.ds(0, BLK)
        pltpu.make_async_copy(emb.at[ia.at[sl]], ab, semr).wait()
        pltpu.make_async_copy(emb.at[ia.at[sl]], tb, semr).wait()
        pltpu.make_async_copy(emb.at[ia.at[sl]], ib, semr).wait()

    def out_start(q, xset):
        xp, semo = xps[xset]
        off = (base0 + q * (GRP * BLK)) * LANES
        pltpu.make_async_copy(xp, out.at[pl.ds(off, GRP * BLK * LANES)], semo).start()

    def out_wait(xset):
        xp, semo = xps[xset]
        pltpu.make_async_copy(
            xp, out.at[pl.ds(base0 * LANES, GRP * BLK * LANES)], semo).wait()

    def compute(blk, rset, xset, xrow):
        ab, tb, ib, _ = rows[rset]
        xp, _ = xps[xset]

        def triplet(j, carry):
            acc1 = jnp.zeros((LANES,), jnp.float32)
            acc12 = jnp.zeros((LANES,), jnp.float32)
            acc2 = jnp.zeros((LANES,), jnp.float32)
            for c in range(CHUNKS):
                sl = pl.ds(c * LANES, LANES)
                av = ab[j, sl]
                tv = tb[j, sl]
                iv = ib[j, sl]
                d1 = tv - av
                d2 = iv - av
                acc1 = acc1 + d1 * d1
                acc12 = acc12 + d1 * d2
                acc2 = acc2 + d2 * d2
            xp[pl.ds((xrow + j) * LANES, LANES)] = 0.5 * acc1 + acc12 - acc2
            return carry

        lax.fori_loop(0, BLK, triplet, 0)

    idx_fetch(0, 0)
    idx_fetch(1, 1)
    idx_wait(0)
    fire(0, 0, 0)
    fire(1, 0, 1)
    drain(0)
    compute(0, 0, 0, 0 * BLK)
    idx_wait(1)
    fire(0, 1, 0)
    drain(1)
    idx_fetch(2, 0)
    compute(1, 1, 0, 1 * BLK)
    fire(1, 1, 1)
    drain(0)
    compute(2, 0, 0, 2 * BLK)
    idx_wait(0)
    fire(0, 0, 0)
    drain(1)
    idx_fetch(3, 1)
    compute(3, 1, 0, 3 * BLK)
    out_start(0, 0)

    def outer(t, carry):
        q1 = 2 * t + 1
        b0 = q1 * GRP

        @pl.when(t > 0)
        def _():
            out_wait(1)

        fire(1, 0, 1)
        drain(0)
        compute(b0 + 0, 0, 1, 0 * BLK)
        idx_wait(1)
        fire(0, 1, 0)
        drain(1)
        idx_fetch(4 * t + 4, 0)
        compute(b0 + 1, 1, 1, 1 * BLK)
        fire(1, 1, 1)
        drain(0)
        compute(b0 + 2, 0, 1, 2 * BLK)
        idx_wait(0)
        fire(0, 0, 0)
        drain(1)
        idx_fetch(4 * t + 5, 1)
        compute(b0 + 3, 1, 1, 3 * BLK)
        out_start(q1, 1)

        q2 = q1 + 1
        c0 = q2 * GRP
        out_wait(0)
        fire(1, 0, 1)
        drain(0)
        compute(c0 + 0, 0, 0, 0 * BLK)
        idx_wait(1)
        fire(0, 1, 0)
        drain(1)

        @pl.when(4 * t + 6 < NPAIR)
        def _():
            idx_fetch(4 * t + 6, 0)

        compute(c0 + 1, 1, 0, 1 * BLK)
        fire(1, 1, 1)
        drain(0)
        compute(c0 + 2, 0, 0, 2 * BLK)
        @pl.when(c0 + 4 < NB_PER)
        def _():
            idx_wait(0)
            fire(0, 0, 0)

        drain(1)

        @pl.when(4 * t + 7 < NPAIR)
        def _():
            idx_fetch(4 * t + 7, 1)

        compute(c0 + 3, 1, 0, 3 * BLK)
        out_start(q2, 0)
        return carry

    lax.fori_loop(0, (NB_PER // GRP - 1) // 2, outer, 0)
    out_wait(1)
    out_wait(0)


def _tc_reduce_body(x_ref, out_ref):
    x = x_ref[...]
    g = lax.broadcasted_iota(jnp.int32, (256, 16), 0) // 16 == \
        lax.broadcasted_iota(jnp.int32, (256, 16), 1)
    s = jax.lax.dot_general(x, g.astype(jnp.float32),
                            (((1,), (0,)), ((), ())),
                            preferred_element_type=jnp.float32)
    sp = jnp.maximum(s, 0.0) + jnp.log1p(jnp.exp(-jnp.abs(s)))
    ids = lax.broadcasted_iota(jnp.int32, (N_PAD // 16, 16), 0) * 16 + \
        lax.broadcasted_iota(jnp.int32, (N_PAD // 16, 16), 1)
    total = jnp.sum(jnp.where(ids < N_TRIP, sp, 0.0), keepdims=True)
    out_ref[...] = total.reshape(1, 1) / N_TRIP


_tc_reduce = pl.pallas_call(
    _tc_reduce_body,
    out_shape=jax.ShapeDtypeStruct((1, 1), jnp.float32),
)


def kernel(inputs, targets, T):
    del targets
    t_pad = jnp.pad(T, ((0, 0), (0, N_PAD - T.shape[1])))
    xpart = _sc_partials(inputs, t_pad[0], t_pad[1], t_pad[2])
    return _tc_reduce(xpart.reshape(N_PAD // 16, 256))[0, 0]


# speedup vs baseline: 3.0693x; 1.0450x over previous
import functools

import jax
import jax.numpy as jnp
from jax import lax
from jax.experimental import pallas as pl
from jax.experimental.pallas import tpu as pltpu
from jax.experimental.pallas import tpu_sc as plsc

N_EMB = 100000
D = 128
N_TRIP = 100000

NC = 2
NS = 16
NW = NC * NS
BLK = 112
GRP = 4
NB_PER = 28
NPAIR = NB_PER // 2
N_PAD = NW * NB_PER * BLK
LANES = 16
CHUNKS = D // LANES

_mesh = plsc.VectorSubcoreMesh(core_axis_name="c", subcore_axis_name="s")


@functools.partial(
    pl.kernel,
    mesh=_mesh,
    out_type=jax.ShapeDtypeStruct((N_PAD * LANES,), jnp.float32),
    scratch_types=[
        pltpu.VMEM((2 * BLK,), jnp.int32),
        pltpu.VMEM((2 * BLK,), jnp.int32),
        pltpu.VMEM((2 * BLK,), jnp.int32),
        pltpu.VMEM((2 * BLK,), jnp.int32),
        pltpu.VMEM((2 * BLK,), jnp.int32),
        pltpu.VMEM((2 * BLK,), jnp.int32),
        pltpu.VMEM((BLK, D), jnp.float32),
        pltpu.VMEM((BLK, D), jnp.float32),
        pltpu.VMEM((BLK, D), jnp.float32),
        pltpu.VMEM((BLK, D), jnp.float32),
        pltpu.VMEM((BLK, D), jnp.float32),
        pltpu.VMEM((BLK, D), jnp.float32),
        pltpu.VMEM((GRP * BLK * LANES,), jnp.float32),
        pltpu.VMEM((GRP * BLK * LANES,), jnp.float32),
        pltpu.SemaphoreType.DMA,
        pltpu.SemaphoreType.DMA,
        pltpu.SemaphoreType.DMA,
        pltpu.SemaphoreType.DMA,
        pltpu.SemaphoreType.DMA,
        pltpu.SemaphoreType.DMA,
    ],
)
def _sc_partials(emb, t0, t1, t2, out,
                 ia0, it0, ii0, ia1, it1, ii1,
                 a0, tb0, ib0, a1, tb1, ib1,
                 xpa, xpb, semr0, semr1, si0, si1, semoa, semob):
    wid = lax.axis_index("s") * NC + lax.axis_index("c")
    base0 = wid * (NB_PER * BLK)

    idxs = ((ia0, it0, ii0, si0), (ia1, it1, ii1, si1))
    rows = ((a0, tb0, ib0, semr0), (a1, tb1, ib1, semr1))
    xps = ((xpa, semoa), (xpb, semob))

    def idx_fetch(pair, iset):
        ia, it, ii, si = idxs[iset]
        off = base0 + pair * (2 * BLK)
        pltpu.make_async_copy(t0.at[pl.ds(off, 2 * BLK)], ia, si).start()
        pltpu.make_async_copy(t1.at[pl.ds(off, 2 * BLK)], it, si).start()
        pltpu.make_async_copy(t2.at[pl.ds(off, 2 * BLK)], ii, si).start()

    def idx_wait(iset):
        ia, it, ii, si = idxs[iset]
        pltpu.make_async_copy(t0.at[pl.ds(base0, 2 * BLK)], ia, si).wait()
        pltpu.make_async_copy(t0.at[pl.ds(base0, 2 * BLK)], it, si).wait()
        pltpu.make_async_copy(t0.at[pl.ds(base0, 2 * BLK)], ii, si).wait()

    H = BLK // 2

    def fire(rset, iset, half):
        ia, it, ii, _ = idxs[iset]
        ab, tb, ib, semr = rows[rset]
        sl0 = pl.ds(half * BLK, H)
        sl1 = pl.ds(half * BLK + H, H)
        pltpu.make_async_copy(emb.at[ia.at[sl0]], ab.at[pl.ds(0, H), :], semr).start()
        pltpu.make_async_copy(emb.at[it.at[sl0]], tb.at[pl.ds(0, H), :], semr).start()
        pltpu.make_async_copy(emb.at[ii.at[sl0]], ib.at[pl.ds(0, H), :], semr).start()
        pltpu.make_async_copy(emb.at[ia.at[sl1]], ab.at[pl.ds(H, H), :], semr).start()
        pltpu.make_async_copy(emb.at[it.at[sl1]], tb.at[pl.ds(H, H), :], semr).start()
        pltpu.make_async_copy(emb.at[ii.at[sl1]], ib.at[pl.ds(H, H), :], semr).start()

    def drain(rset):
        ia, _, _, _ = idxs[0]
        ab, tb, ib, semr = rows[rset]
        sl = pl.ds(0, H)
        for dst in (ab, tb, ib):
            pltpu.make_async_copy(
                emb.at[ia.at[sl]], dst.at[pl.ds(0, H), :], semr).wait()
            pltpu.make_async_copy(
                emb.at[ia.at[sl]], dst.at[pl.ds(H, H), :], semr).wait()

    def out_start(q, xset):
        xp, semo = xps[xset]
        off = (base0 + q * (GRP * BLK)) * LANES
        pltpu.make_async_copy(xp, out.at[pl.ds(off, GRP * BLK * LANES)], semo).start()

    def out_wait(xset):
        xp, semo = xps[xset]
        pltpu.make_async_copy(
            xp, out.at[pl.ds(base0 * LANES, GRP * BLK * LANES)], semo).wait()

    def compute(blk, rset, xset, xrow):
        ab, tb, ib, _ = rows[rset]
        xp, _ = xps[xset]

        def triplet(j, carry):
            acc1 = jnp.zeros((LANES,), jnp.float32)
            acc12 = jnp.zeros((LANES,), jnp.float32)
            acc2 = jnp.zeros((LANES,), jnp.float32)
            for c in range(CHUNKS):
                sl = pl.ds(c * LANES, LANES)
                av = ab[j, sl]
                tv = tb[j, sl]
                iv = ib[j, sl]
                d1 = tv - av
                d2 = iv - av
                acc1 = acc1 + d1 * d1
                acc12 = acc12 + d1 * d2
                acc2 = acc2 + d2 * d2
            xp[pl.ds((xrow + j) * LANES, LANES)] = 0.5 * acc1 + acc12 - acc2
            return carry

        lax.fori_loop(0, BLK, triplet, 0)

    idx_fetch(0, 0)
    idx_fetch(1, 1)
    idx_wait(0)
    fire(0, 0, 0)
    fire(1, 0, 1)
    drain(0)
    compute(0, 0, 0, 0 * BLK)
    idx_wait(1)
    fire(0, 1, 0)
    drain(1)
    idx_fetch(2, 0)
    compute(1, 1, 0, 1 * BLK)
    fire(1, 1, 1)
    drain(0)
    compute(2, 0, 0, 2 * BLK)
    idx_wait(0)
    fire(0, 0, 0)
    drain(1)
    idx_fetch(3, 1)
    compute(3, 1, 0, 3 * BLK)
    out_start(0, 0)

    def outer(t, carry):
        q1 = 2 * t + 1
        b0 = q1 * GRP

        @pl.when(t > 0)
        def _():
            out_wait(1)

        fire(1, 0, 1)
        drain(0)
        compute(b0 + 0, 0, 1, 0 * BLK)
        idx_wait(1)
        fire(0, 1, 0)
        drain(1)
        idx_fetch(4 * t + 4, 0)
        compute(b0 + 1, 1, 1, 1 * BLK)
        fire(1, 1, 1)
        drain(0)
        compute(b0 + 2, 0, 1, 2 * BLK)
        idx_wait(0)
        fire(0, 0, 0)
        drain(1)
        idx_fetch(4 * t + 5, 1)
        compute(b0 + 3, 1, 1, 3 * BLK)
        out_start(q1, 1)

        q2 = q1 + 1
        c0 = q2 * GRP
        out_wait(0)
        fire(1, 0, 1)
        drain(0)
        compute(c0 + 0, 0, 0, 0 * BLK)
        idx_wait(1)
        fire(0, 1, 0)
        drain(1)

        @pl.when(4 * t + 6 < NPAIR)
        def _():
            idx_fetch(4 * t + 6, 0)

        compute(c0 + 1, 1, 0, 1 * BLK)
        fire(1, 1, 1)
        drain(0)
        compute(c0 + 2, 0, 0, 2 * BLK)
        @pl.when(c0 + 4 < NB_PER)
        def _():
            idx_wait(0)
            fire(0, 0, 0)

        drain(1)

        @pl.when(4 * t + 7 < NPAIR)
        def _():
            idx_fetch(4 * t + 7, 1)

        compute(c0 + 3, 1, 0, 3 * BLK)
        out_start(q2, 0)
        return carry

    lax.fori_loop(0, (NB_PER // GRP - 1) // 2, outer, 0)
    out_wait(1)
    out_wait(0)


def _tc_reduce_body(x_ref, out_ref):
    x = x_ref[...].reshape(N_PAD // 16, 256)
    g = lax.broadcasted_iota(jnp.int32, (256, 16), 0) // 16 == \
        lax.broadcasted_iota(jnp.int32, (256, 16), 1)
    s = jax.lax.dot_general(x, g.astype(jnp.float32),
                            (((1,), (0,)), ((), ())),
                            preferred_element_type=jnp.float32)
    sp = jnp.maximum(s, 0.0) + jnp.log1p(jnp.exp(-jnp.abs(s)))
    ids = lax.broadcasted_iota(jnp.int32, (N_PAD // 16, 16), 0) * 16 + \
        lax.broadcasted_iota(jnp.int32, (N_PAD // 16, 16), 1)
    total = jnp.sum(jnp.where(ids < N_TRIP, sp, 0.0), keepdims=True)
    out_ref[...] = total.reshape(1, 1) / N_TRIP


_tc_reduce = pl.pallas_call(
    _tc_reduce_body,
    out_shape=jax.ShapeDtypeStruct((1, 1), jnp.float32),
)


def kernel(inputs, targets, T):
    del targets
    t_pad = jnp.pad(T, ((0, 0), (0, N_PAD - T.shape[1])))
    xpart = _sc_partials(inputs, t_pad[0], t_pad[1], t_pad[2])
    return _tc_reduce(xpart)[0, 0]


# speedup vs baseline: 3.1902x; 1.0394x over previous
import functools

import jax
import jax.numpy as jnp
from jax import lax
from jax.experimental import pallas as pl
from jax.experimental.pallas import tpu as pltpu
from jax.experimental.pallas import tpu_sc as plsc

N_EMB = 100000
D = 128
N_TRIP = 100000

NC = 2
NS = 16
NW = NC * NS
BLK = 112
GRP = 4
NB_A = 36
NB_B = 20
NB_SUM = NB_A + NB_B
N_PAD = NS * NB_SUM * BLK
LANES = 16
CHUNKS = D // LANES

_mesh = plsc.VectorSubcoreMesh(core_axis_name="c", subcore_axis_name="s")


@functools.partial(
    pl.kernel,
    mesh=_mesh,
    out_type=jax.ShapeDtypeStruct((N_PAD * LANES,), jnp.float32),
    scratch_types=[
        pltpu.VMEM((2 * BLK,), jnp.int32),
        pltpu.VMEM((2 * BLK,), jnp.int32),
        pltpu.VMEM((2 * BLK,), jnp.int32),
        pltpu.VMEM((2 * BLK,), jnp.int32),
        pltpu.VMEM((2 * BLK,), jnp.int32),
        pltpu.VMEM((2 * BLK,), jnp.int32),
        pltpu.VMEM((BLK, D), jnp.float32),
        pltpu.VMEM((BLK, D), jnp.float32),
        pltpu.VMEM((BLK, D), jnp.float32),
        pltpu.VMEM((BLK, D), jnp.float32),
        pltpu.VMEM((BLK, D), jnp.float32),
        pltpu.VMEM((BLK, D), jnp.float32),
        pltpu.VMEM((GRP * BLK * LANES,), jnp.float32),
        pltpu.VMEM((GRP * BLK * LANES,), jnp.float32),
        pltpu.SemaphoreType.DMA,
        pltpu.SemaphoreType.DMA,
        pltpu.SemaphoreType.DMA,
        pltpu.SemaphoreType.DMA,
        pltpu.SemaphoreType.DMA,
        pltpu.SemaphoreType.DMA,
    ],
)
def _sc_partials(emb, t0, t1, t2, out,
                 ia0, it0, ii0, ia1, it1, ii1,
                 a0, tb0, ib0, a1, tb1, ib1,
                 xpa, xpb, semr0, semr1, si0, si1, semoa, semob):
    cbit = lax.axis_index("c")
    sid = lax.axis_index("s")
    base0 = (sid * NB_SUM + cbit * NB_A) * BLK

    idxs = ((ia0, it0, ii0, si0), (ia1, it1, ii1, si1))
    rows = ((a0, tb0, ib0, semr0), (a1, tb1, ib1, semr1))
    xps = ((xpa, semoa), (xpb, semob))

    def idx_fetch(pair, iset):
        ia, it, ii, si = idxs[iset]
        off = base0 + pair * (2 * BLK)
        pltpu.make_async_copy(t0.at[pl.ds(off, 2 * BLK)], ia, si).start()
        pltpu.make_async_copy(t1.at[pl.ds(off, 2 * BLK)], it, si).start()
        pltpu.make_async_copy(t2.at[pl.ds(off, 2 * BLK)], ii, si).start()

    def idx_wait(iset):
        ia, it, ii, si = idxs[iset]
        pltpu.make_async_copy(t0.at[pl.ds(base0, 2 * BLK)], ia, si).wait()
        pltpu.make_async_copy(t0.at[pl.ds(base0, 2 * BLK)], it, si).wait()
        pltpu.make_async_copy(t0.at[pl.ds(base0, 2 * BLK)], ii, si).wait()

    H = BLK // 2

    def fire(rset, iset, half):
        ia, it, ii, _ = idxs[iset]
        ab, tb, ib, semr = rows[rset]
        sl0 = pl.ds(half * BLK, H)
        sl1 = pl.ds(half * BLK + H, H)
        pltpu.make_async_copy(emb.at[ia.at[sl0]], ab.at[pl.ds(0, H), :], semr).start()
        pltpu.make_async_copy(emb.at[it.at[sl0]], tb.at[pl.ds(0, H), :], semr).start()
        pltpu.make_async_copy(emb.at[ii.at[sl0]], ib.at[pl.ds(0, H), :], semr).start()
        pltpu.make_async_copy(emb.at[ia.at[sl1]], ab.at[pl.ds(H, H), :], semr).start()
        pltpu.make_async_copy(emb.at[it.at[sl1]], tb.at[pl.ds(H, H), :], semr).start()
        pltpu.make_async_copy(emb.at[ii.at[sl1]], ib.at[pl.ds(H, H), :], semr).start()

    def drain(rset):
        ia, _, _, _ = idxs[0]
        ab, tb, ib, semr = rows[rset]
        sl = pl.ds(0, H)
        for dst in (ab, tb, ib):
            pltpu.make_async_copy(
                emb.at[ia.at[sl]], dst.at[pl.ds(0, H), :], semr).wait()
            pltpu.make_async_copy(
                emb.at[ia.at[sl]], dst.at[pl.ds(H, H), :], semr).wait()

    def out_start(q, xset):
        xp, semo = xps[xset]
        off = (base0 + q * (GRP * BLK)) * LANES
        pltpu.make_async_copy(xp, out.at[pl.ds(off, GRP * BLK * LANES)], semo).start()

    def out_wait(xset):
        xp, semo = xps[xset]
        pltpu.make_async_copy(
            xp, out.at[pl.ds(base0 * LANES, GRP * BLK * LANES)], semo).wait()

    def compute(blk, rset, xset, xrow):
        ab, tb, ib, _ = rows[rset]
        xp, _ = xps[xset]

        def triplet(j, carry):
            acc1 = jnp.zeros((LANES,), jnp.float32)
            acc12 = jnp.zeros((LANES,), jnp.float32)
            acc2 = jnp.zeros((LANES,), jnp.float32)
            for c in range(CHUNKS):
                sl = pl.ds(c * LANES, LANES)
                av = ab[j, sl]
                tv = tb[j, sl]
                iv = ib[j, sl]
                d1 = tv - av
                d2 = iv - av
                acc1 = acc1 + d1 * d1
                acc12 = acc12 + d1 * d2
                acc2 = acc2 + d2 * d2
            xp[pl.ds((xrow + j) * LANES, LANES)] = 0.5 * acc1 + acc12 - acc2
            return carry

        lax.fori_loop(0, BLK, triplet, 0)

    def schedule(nb):
        npair = nb // 2
        idx_fetch(0, 0)
        idx_fetch(1, 1)
        idx_wait(0)
        fire(0, 0, 0)
        fire(1, 0, 1)
        drain(0)
        compute(0, 0, 0, 0 * BLK)
        idx_wait(1)
        fire(0, 1, 0)
        drain(1)
        idx_fetch(2, 0)
        compute(1, 1, 0, 1 * BLK)
        fire(1, 1, 1)
        drain(0)
        compute(2, 0, 0, 2 * BLK)
        idx_wait(0)
        fire(0, 0, 0)
        drain(1)
        idx_fetch(3, 1)
        compute(3, 1, 0, 3 * BLK)
        out_start(0, 0)

        def outer(t, carry):
            q1 = 2 * t + 1
            b0 = q1 * GRP

            @pl.when(t > 0)
            def _():
                out_wait(1)

            fire(1, 0, 1)
            drain(0)
            compute(b0 + 0, 0, 1, 0 * BLK)
            idx_wait(1)
            fire(0, 1, 0)
            drain(1)
            idx_fetch(4 * t + 4, 0)
            compute(b0 + 1, 1, 1, 1 * BLK)
            fire(1, 1, 1)
            drain(0)
            compute(b0 + 2, 0, 1, 2 * BLK)
            idx_wait(0)
            fire(0, 0, 0)
            drain(1)
            idx_fetch(4 * t + 5, 1)
            compute(b0 + 3, 1, 1, 3 * BLK)
            out_start(q1, 1)

            q2 = q1 + 1
            c0 = q2 * GRP
            out_wait(0)
            fire(1, 0, 1)
            drain(0)
            compute(c0 + 0, 0, 0, 0 * BLK)
            idx_wait(1)
            fire(0, 1, 0)
            drain(1)

            @pl.when(4 * t + 6 < npair)
            def _():
                idx_fetch(4 * t + 6, 0)

            compute(c0 + 1, 1, 0, 1 * BLK)
            fire(1, 1, 1)
            drain(0)
            compute(c0 + 2, 0, 0, 2 * BLK)
            @pl.when(c0 + 4 < nb)
            def _():
                idx_wait(0)
                fire(0, 0, 0)

            drain(1)

            @pl.when(4 * t + 7 < npair)
            def _():
                idx_fetch(4 * t + 7, 1)

            compute(c0 + 3, 1, 0, 3 * BLK)
            out_start(q2, 0)
            return carry

        lax.fori_loop(0, (nb // GRP - 1) // 2, outer, 0)
        out_wait(1)
        out_wait(0)

    @pl.when(cbit == 0)
    def _():
        schedule(NB_A)

    @pl.when(cbit == 1)
    def _():
        schedule(NB_B)


def _tc_reduce_body(x_ref, out_ref):
    x = x_ref[...].reshape(N_PAD // 16, 256)
    g = lax.broadcasted_iota(jnp.int32, (256, 16), 0) // 16 == \
        lax.broadcasted_iota(jnp.int32, (256, 16), 1)
    s = jax.lax.dot_general(x, g.astype(jnp.float32),
                            (((1,), (0,)), ((), ())),
                            preferred_element_type=jnp.float32)
    sp = jnp.maximum(s, 0.0) + jnp.log1p(jnp.exp(-jnp.abs(s)))
    ids = lax.broadcasted_iota(jnp.int32, (N_PAD // 16, 16), 0) * 16 + \
        lax.broadcasted_iota(jnp.int32, (N_PAD // 16, 16), 1)
    total = jnp.sum(jnp.where(ids < N_TRIP, sp, 0.0), keepdims=True)
    out_ref[...] = total.reshape(1, 1) / N_TRIP


_tc_reduce = pl.pallas_call(
    _tc_reduce_body,
    out_shape=jax.ShapeDtypeStruct((1, 1), jnp.float32),
)


def kernel(inputs, targets, T):
    del targets
    t_pad = jnp.pad(T, ((0, 0), (0, N_PAD - T.shape[1])))
    xpart = _sc_partials(inputs, t_pad[0], t_pad[1], t_pad[2])
    return _tc_reduce(xpart)[0, 0]
